# baseline (speedup 1.0000x reference)
import jax
import jax.numpy as jnp
from jax import lax
from jax.experimental import pallas as pl
from jax.experimental.pallas import tpu as pltpu
from jax.experimental.pallas import tpu_sc as plsc

D_MODEL = 768
BATCH = 4
SEQ_LEN = 2048

NUM_CORES = 2
NUM_SUBCORES = 16
NUM_WORKERS = NUM_CORES * NUM_SUBCORES
POS_PER_W = SEQ_LEN // NUM_WORKERS
LANES = 16


def _sc_embed_body(src_hbm, tok_hbm, pos_hbm, out_hbm, idx_v, pos_v, tok_v,
                   gsem):
    cid = lax.axis_index("c")
    sid = lax.axis_index("s")
    wid = sid * NUM_CORES + cid
    p0 = wid * POS_PER_W

    pltpu.sync_copy(pos_hbm.at[pl.ds(p0, POS_PER_W)], pos_v)

    def _batch(b, carry):
        pltpu.sync_copy(src_hbm.at[b, pl.ds(p0, POS_PER_W)], idx_v)
        pltpu.async_copy(tok_hbm.at[idx_v], tok_v, gsem).wait()

        def _row_add(r, inner):
            for j in range(D_MODEL // LANES):
                sl = pl.ds(j * LANES, LANES)
                plsc.addupdate(tok_v.at[r, sl], pos_v[r, sl])
            return inner

        lax.fori_loop(0, POS_PER_W, _row_add, 0)
        pltpu.sync_copy(tok_v, out_hbm.at[b, pl.ds(p0, POS_PER_W)])
        return carry

    lax.fori_loop(0, BATCH, _batch, 0)


@jax.jit
def _sc_embed(src, token_table, pos_table):
    mesh = plsc.VectorSubcoreMesh(
        core_axis_name="c",
        subcore_axis_name="s",
        num_cores=NUM_CORES,
        num_subcores=NUM_SUBCORES,
    )
    f = pl.kernel(
        _sc_embed_body,
        out_type=jax.ShapeDtypeStruct((BATCH, SEQ_LEN, D_MODEL), jnp.float32),
        mesh=mesh,
        scratch_types=[
            pltpu.VMEM((POS_PER_W,), jnp.int32),
            pltpu.VMEM((POS_PER_W, D_MODEL), jnp.float32),
            pltpu.VMEM((POS_PER_W, D_MODEL), jnp.float32),
            pltpu.SemaphoreType.DMA,
        ],
    )
    return f(src, token_table, pos_table)


def kernel(src, token_table, pos_table):
    return _sc_embed(src.astype(jnp.int32), token_table, pos_table)


# speedup vs baseline: 1.0254x; 1.0254x over previous
import jax
import jax.numpy as jnp
from jax import lax
from jax.experimental import pallas as pl
from jax.experimental.pallas import tpu as pltpu
from jax.experimental.pallas import tpu_sc as plsc

D_MODEL = 768
BATCH = 4
SEQ_LEN = 2048

NUM_CORES = 2
NUM_SUBCORES = 16
NUM_WORKERS = NUM_CORES * NUM_SUBCORES
POS_PER_W = SEQ_LEN // NUM_WORKERS
LANES = 16


def _sc_embed_body(src_hbm, tok_hbm, pos_hbm, out_hbm, idx_v, pos_v, tok_v,
                   gsem):
    cid = lax.axis_index("c")
    sid = lax.axis_index("s")
    wid = sid * NUM_CORES + cid
    p0 = wid * POS_PER_W

    idx_copies = [
        pltpu.async_copy(src_hbm.at[b, pl.ds(p0, POS_PER_W)], idx_v.at[b], gsem)
        for b in range(BATCH)
    ]
    pltpu.sync_copy(pos_hbm.at[pl.ds(p0, POS_PER_W)], pos_v)
    for cpy in idx_copies:
        cpy.wait()

    def _batch(b, carry):
        pltpu.async_copy(tok_hbm.at[idx_v.at[b]], tok_v, gsem).wait()

        def _row_add(r, inner):
            for j in range(D_MODEL // LANES):
                sl = pl.ds(j * LANES, LANES)
                plsc.addupdate(tok_v.at[r, sl], pos_v[r, sl])
            return inner

        lax.fori_loop(0, POS_PER_W, _row_add, 0)
        pltpu.sync_copy(tok_v, out_hbm.at[b, pl.ds(p0, POS_PER_W)])
        return carry

    lax.fori_loop(0, BATCH, _batch, 0)


@jax.jit
def _sc_embed(src, token_table, pos_table):
    mesh = plsc.VectorSubcoreMesh(
        core_axis_name="c",
        subcore_axis_name="s",
        num_cores=NUM_CORES,
        num_subcores=NUM_SUBCORES,
    )
    f = pl.kernel(
        _sc_embed_body,
        out_type=jax.ShapeDtypeStruct((BATCH, SEQ_LEN, D_MODEL), jnp.float32),
        mesh=mesh,
        scratch_types=[
            pltpu.VMEM((BATCH, POS_PER_W), jnp.int32),
            pltpu.VMEM((POS_PER_W, D_MODEL), jnp.float32),
            pltpu.VMEM((POS_PER_W, D_MODEL), jnp.float32),
            pltpu.SemaphoreType.DMA,
        ],
    )
    return f(src, token_table, pos_table)


def kernel(src, token_table, pos_table):
    return _sc_embed(src.astype(jnp.int32), token_table, pos_table)
